# baseline (speedup 1.0000x reference)
import functools

import jax
import jax.numpy as jnp
from jax import lax
from jax.experimental import pallas as pl
from jax.experimental.pallas import tpu as pltpu
from jax.experimental.pallas import tpu_sc as plsc

VOCAB = 32
D = 8
B, T = 4096, 200
NC, NS, L = 2, 16, 16
NW = NC * NS
BIG = T - (T // NW) * NW
SLABS_BIG = T // NW + 1
SLABS_SMALL = T // NW

_mesh = plsc.VectorSubcoreMesh(core_axis_name="c", subcore_axis_name="s")


@functools.partial(
    pl.kernel,
    mesh=_mesh,
    out_type=jax.ShapeDtypeStruct((T, D, B), jnp.float32),
    scratch_types=[
        pltpu.VMEM((VOCAB * D,), jnp.float32),
        pltpu.VMEM((B,), jnp.int32),
        pltpu.VMEM((D, B), jnp.float32),
    ],
    compiler_params=pltpu.CompilerParams(needs_layout_passes=False),
)
def _embed_lookup(ids_hbm, table_hbm, out_hbm, table_v, idx_v, slab_v):
    wid = lax.axis_index("s") * NC + lax.axis_index("c")
    t0 = jnp.where(wid < BIG, wid * SLABS_BIG,
                   BIG * SLABS_BIG + (wid - BIG) * SLABS_SMALL)
    nt = jnp.where(wid < BIG, SLABS_BIG, SLABS_SMALL)

    pltpu.sync_copy(table_hbm, table_v)

    def slab_body(k, carry):
        t = t0 + k
        pltpu.sync_copy(ids_hbm.at[pl.ds(t * B, B)], idx_v)

        @plsc.parallel_loop(0, B, step=L, unroll=16)
        def group_body(i):
            o = pl.multiple_of(i, L)
            gb = idx_v[pl.ds(o, L)] * D
            for j in range(D):
                slab_v[j, pl.ds(o, L)] = plsc.load_gather(table_v, [gb + j])

        pltpu.sync_copy(slab_v, out_hbm.at[t])
        return carry

    lax.fori_loop(0, nt, slab_body, 0)


def kernel(input_ids, embed_tokens):
    ids_t = input_ids.T.reshape(-1).astype(jnp.int32)
    out = _embed_lookup(ids_t, embed_tokens.reshape(-1))
    return out.transpose(2, 0, 1)


# speedup vs baseline: 1.1200x; 1.1200x over previous
import functools

import jax
import jax.numpy as jnp
from jax import lax
from jax.experimental import pallas as pl
from jax.experimental.pallas import tpu as pltpu
from jax.experimental.pallas import tpu_sc as plsc

VOCAB = 32
D = 8
B, T = 4096, 200
NC, NS, L = 2, 16, 16
NW = NC * NS
BIG = T - (T // NW) * NW
SLABS_BIG = T // NW + 1
SLABS_SMALL = T // NW

_mesh = plsc.VectorSubcoreMesh(core_axis_name="c", subcore_axis_name="s")


@functools.partial(
    pl.kernel,
    mesh=_mesh,
    out_type=jax.ShapeDtypeStruct((T, D, B), jnp.float32),
    scratch_types=[
        pltpu.VMEM((VOCAB * D,), jnp.float32),
        pltpu.VMEM((2, B), jnp.int32),
        pltpu.VMEM((2, D, B), jnp.float32),
        pltpu.SemaphoreType.DMA((2,)),
        pltpu.SemaphoreType.DMA((2,)),
    ],
    compiler_params=pltpu.CompilerParams(needs_layout_passes=False),
)
def _embed_lookup(ids_hbm, table_hbm, out_hbm, table_v, idx2, slab2, sem_in, sem_out):
    wid = lax.axis_index("s") * NC + lax.axis_index("c")
    t0 = jnp.where(wid < BIG, wid * SLABS_BIG,
                   BIG * SLABS_BIG + (wid - BIG) * SLABS_SMALL)
    nt = jnp.where(wid < BIG, SLABS_BIG, SLABS_SMALL)

    pltpu.sync_copy(table_hbm, table_v)

    def in_cp(k):
        return pltpu.make_async_copy(
            ids_hbm.at[pl.ds((t0 + k) * B, B)], idx2.at[k % 2], sem_in.at[k % 2])

    def out_cp(k):
        return pltpu.make_async_copy(
            slab2.at[k % 2], out_hbm.at[t0 + k], sem_out.at[k % 2])

    in_cp(0).start()

    def do_slab(k):
        in_cp(k).wait()
        if k + 1 < SLABS_BIG:
            @pl.when(k + 1 < nt)
            def _():
                in_cp(k + 1).start()
        if k >= 1:
            out_cp(k - 1).wait()
        buf = k % 2

        @plsc.parallel_loop(0, B, step=L, unroll=8)
        def group_body(i):
            o = pl.multiple_of(i, L)
            gb = idx2[buf, pl.ds(o, L)] * D
            for j in range(D):
                slab2[buf, j, pl.ds(o, L)] = plsc.load_gather(table_v, [gb + j])

        out_cp(k).start()

    for k in range(SLABS_BIG):
        if k < SLABS_SMALL:
            do_slab(k)
        else:
            pl.when(k < nt)(lambda: do_slab(k))

    @pl.when(nt == SLABS_SMALL)
    def _():
        out_cp(SLABS_SMALL - 1).wait()

    @pl.when(nt == SLABS_BIG)
    def _():
        out_cp(SLABS_BIG - 1).wait()


def kernel(input_ids, embed_tokens):
    ids_t = input_ids.T.reshape(-1).astype(jnp.int32)
    out = _embed_lookup(ids_t, embed_tokens.reshape(-1))
    return out.transpose(2, 0, 1)


# speedup vs baseline: 1.2451x; 1.1117x over previous
import functools

import jax
import jax.numpy as jnp
from jax import lax
from jax.experimental import pallas as pl
from jax.experimental.pallas import tpu as pltpu
from jax.experimental.pallas import tpu_sc as plsc

VOCAB = 32
D = 8
B, T = 4096, 200
NC, NS, L = 2, 16, 16
NW = NC * NS
BIG = T - (T // NW) * NW
SLABS_BIG = T // NW + 1
SLABS_SMALL = T // NW

_mesh = plsc.VectorSubcoreMesh(core_axis_name="c", subcore_axis_name="s")


@functools.partial(
    pl.kernel,
    mesh=_mesh,
    out_type=jax.ShapeDtypeStruct((T, D, B), jnp.float32),
    scratch_types=[
        pltpu.VMEM((VOCAB * D,), jnp.float32),
        pltpu.VMEM((2, B), jnp.int32),
        pltpu.VMEM((2, D, B), jnp.float32),
        pltpu.SemaphoreType.DMA((2,)),
        pltpu.SemaphoreType.DMA((2,)),
    ],
    compiler_params=pltpu.CompilerParams(needs_layout_passes=False),
)
def _embed_lookup(ids_hbm, table_hbm, out_hbm, table_v, idx2, slab2, sem_in, sem_out):
    wid = lax.axis_index("s") * NC + lax.axis_index("c")
    t0 = jnp.where(wid < BIG, wid * SLABS_BIG,
                   BIG * SLABS_BIG + (wid - BIG) * SLABS_SMALL)
    nt = jnp.where(wid < BIG, SLABS_BIG, SLABS_SMALL)

    pltpu.sync_copy(table_hbm, table_v)

    def in_cp(k):
        return pltpu.make_async_copy(
            ids_hbm.at[pl.ds((t0 + k) * B, B)], idx2.at[k % 2], sem_in.at[k % 2])

    def out_cp(k):
        return pltpu.make_async_copy(
            slab2.at[k % 2], out_hbm.at[t0 + k], sem_out.at[k % 2])

    in_cp(0).start()

    def do_slab(k):
        in_cp(k).wait()
        if k + 1 < SLABS_BIG:
            @pl.when(k + 1 < nt)
            def _():
                in_cp(k + 1).start()
        buf = k % 2

        @plsc.parallel_loop(0, B, step=L, unroll=8)
        def group_body(i):
            o = pl.multiple_of(i, L)
            gb = idx2[buf, pl.ds(o, L)] * D
            for j in range(D):
                slab2[buf, j, pl.ds(o, L)] = plsc.load_gather(table_v, [gb + j])

        if k >= 1:
            out_cp(k - 1).wait()
        out_cp(k).start()

    for k in range(SLABS_BIG):
        if k < SLABS_SMALL:
            do_slab(k)
        else:
            pl.when(k < nt)(lambda: do_slab(k))

    @pl.when(nt == SLABS_SMALL)
    def _():
        out_cp(SLABS_SMALL - 1).wait()

    @pl.when(nt == SLABS_BIG)
    def _():
        out_cp(SLABS_BIG - 1).wait()


def kernel(input_ids, embed_tokens):
    ids_t = input_ids.T.reshape(-1).astype(jnp.int32)
    out = _embed_lookup(ids_t, embed_tokens.reshape(-1))
    return out.transpose(2, 0, 1)
